# baseline (speedup 1.0000x reference)
import functools

import jax
import jax.numpy as jnp
from jax import lax
from jax.experimental import pallas as pl
from jax.experimental.pallas import tpu as pltpu
from jax.experimental.pallas import tpu_sc as plsc

N_S = 10000
N_H = 10000
DIM = 128
RANK = 64
LAYER = 2
TEMP = 0.2
LAMBDA_1 = 0.2
LAMBDA_2 = 1e-07
BN_EPS = 1e-05
B = 4096
BU = 1024

E_EDGES = 320000
CHUNK = 112
CHUNKS_PER_TILE = 180
N_CHUNKS = CHUNKS_PER_TILE * 16
E_PAD = N_CHUNKS * CHUNK
ROWS_MAIN = 624
ROWS_LAST = N_S - 15 * ROWS_MAIN

K_PAD = 10240
K_TILE = 1024

_MESH = plsc.VectorSubcoreMesh(core_axis_name="c", subcore_axis_name="s")


def _fill_zeros(rbuf):
    zeros16 = jnp.zeros((16,), jnp.float32)

    def body(r, _):
        for d in range(8):
            rbuf[r, pl.ds(d * 16, 16)] = zeros16
        return 0

    lax.fori_loop(0, CHUNK, body, 0)


def _do_spmm(packed, pvals, x_hbm, z_hbm, chunk_base, z_row_base,
             acc, ibufs, vbufs, rbufs, sidxs, svals,
             isems, vsems, gsems, ssems, sid):
    base_row = sid * ROWS_MAIN
    rbuf0 = rbufs[0]

    _fill_zeros(rbuf0)

    @pl.when(sid < 15)
    def _():
        for k in range(5):
            pltpu.sync_copy(rbuf0, acc.at[pl.ds(base_row + k * CHUNK, CHUNK)])
        rem = ROWS_MAIN - 5 * CHUNK
        pltpu.sync_copy(rbuf0.at[pl.ds(0, rem)],
                        acc.at[pl.ds(base_row + 5 * CHUNK, rem)])

    @pl.when(sid == 15)
    def _():
        base = 15 * ROWS_MAIN
        for k in range(5):
            pltpu.sync_copy(rbuf0, acc.at[pl.ds(base + k * CHUNK, CHUNK)])
        rem = ROWS_LAST - 5 * CHUNK
        pltpu.sync_copy(rbuf0.at[pl.ds(0, rem)],
                        acc.at[pl.ds(base + 5 * CHUNK, rem)])

    plsc.subcore_barrier()

    base_chunk = chunk_base + sid * CHUNKS_PER_TILE
    n = CHUNKS_PER_TILE

    def idx_start(j, p2):
        pltpu.async_copy(packed.at[base_chunk + j], ibufs[p2], isems[p2])
        pltpu.async_copy(pvals.at[base_chunk + j], vbufs[p2], vsems[p2])

    def idx_wait(j, p2):
        pltpu.make_async_copy(packed.at[base_chunk + j], ibufs[p2],
                              isems[p2]).wait()
        pltpu.make_async_copy(pvals.at[base_chunk + j], vbufs[p2],
                              vsems[p2]).wait()

    def gather_start(j, p2, p3):
        pltpu.async_copy(x_hbm.at[ibufs[p2].at[0]], rbufs[p3], gsems[p3])

    def gather_wait(p2, p3):
        pltpu.make_async_copy(x_hbm.at[ibufs[p2].at[0]], rbufs[p3],
                              gsems[p3]).wait()

    def scatter_start(p3):
        pltpu.async_copy(rbufs[p3], acc.at[sidxs[p3]], ssems[p3], add=True)

    def scatter_wait(p3):
        pltpu.make_async_copy(rbufs[p3], acc.at[sidxs[p3]],
                              ssems[p3]).wait()

    def half(j, k):
        p3 = k % 3
        p2 = k % 2
        rbuf, sidx, sval = rbufs[p3], sidxs[p3], svals[p3]
        ibuf, vbuf = ibufs[p2], vbufs[p2]

        @pl.when(j + 1 < n)
        def _():
            idx_wait(j + 1, (k + 1) % 2)

        @pl.when(j >= 2)
        def _():
            scatter_wait((k + 1) % 3)

        @pl.when(j + 1 < n)
        def _():
            gather_start(j + 1, (k + 1) % 2, (k + 1) % 3)

        gather_wait(p2, p3)
        for g in range(CHUNK // 16):
            sl = pl.ds(g * 16, 16)
            sidx[sl] = ibuf[1, sl]
            sval[sl] = vbuf[sl]

        @pl.when(j + 2 < n)
        def _():
            idx_start(j + 2, k % 2)

        def scale_body(g, _):
            vv = sval[pl.ds(g * 16, 16)]
            for l in range(16):
                v = vv[l]
                e = g * 16 + l
                for d in range(8):
                    sl = pl.ds(d * 16, 16)
                    rbuf[e, sl] = rbuf[e, sl] * v
            return 0

        lax.fori_loop(0, CHUNK // 16, scale_body, 0)
        scatter_start(p3)

    idx_start(0, 0)
    idx_wait(0, 0)
    gather_start(0, 0, 0)
    idx_start(1, 1)

    def chunk_body(i6, _):
        for k in range(6):
            half(6 * i6 + k, k)
        return 0

    lax.fori_loop(0, n // 6, chunk_body, 0)
    scatter_wait((n - 2) % 3)
    scatter_wait((n - 1) % 3)
    plsc.subcore_barrier()

    zb = z_row_base + base_row

    @pl.when(sid < 15)
    def _():
        for k in range(5):
            pltpu.sync_copy(acc.at[pl.ds(base_row + k * CHUNK, CHUNK)],
                            z_hbm.at[pl.ds(zb + k * CHUNK, CHUNK)])
        rem = ROWS_MAIN - 5 * CHUNK
        pltpu.sync_copy(acc.at[pl.ds(base_row + 5 * CHUNK, rem)],
                        z_hbm.at[pl.ds(zb + 5 * CHUNK, rem)])

    @pl.when(sid == 15)
    def _():
        base = 15 * ROWS_MAIN
        zbase = z_row_base + base
        for k in range(5):
            pltpu.sync_copy(acc.at[pl.ds(base + k * CHUNK, CHUNK)],
                            z_hbm.at[pl.ds(zbase + k * CHUNK, CHUNK)])
        rem = ROWS_LAST - 5 * CHUNK
        pltpu.sync_copy(acc.at[pl.ds(base + 5 * CHUNK, rem)],
                        z_hbm.at[pl.ds(zbase + 5 * CHUNK, rem)])

    plsc.subcore_barrier()


@functools.partial(
    pl.kernel,
    mesh=_MESH,
    out_type=jax.ShapeDtypeStruct((8 * N_S, DIM), jnp.float32),
    scratch_types=[
        pltpu.VMEM_SHARED((N_S, DIM), jnp.float32),
        pltpu.VMEM((2, CHUNK), jnp.int32),
        pltpu.VMEM((2, CHUNK), jnp.int32),
        pltpu.VMEM((CHUNK,), jnp.float32),
        pltpu.VMEM((CHUNK,), jnp.float32),
        pltpu.VMEM((CHUNK, DIM), jnp.float32),
        pltpu.VMEM((CHUNK, DIM), jnp.float32),
        pltpu.VMEM((CHUNK, DIM), jnp.float32),
        pltpu.VMEM((CHUNK,), jnp.int32),
        pltpu.VMEM((CHUNK,), jnp.int32),
        pltpu.VMEM((CHUNK,), jnp.int32),
        pltpu.VMEM((CHUNK,), jnp.float32),
        pltpu.VMEM((CHUNK,), jnp.float32),
        pltpu.VMEM((CHUNK,), jnp.float32),
        pltpu.SemaphoreType.DMA,
        pltpu.SemaphoreType.DMA,
        pltpu.SemaphoreType.DMA,
        pltpu.SemaphoreType.DMA,
        pltpu.SemaphoreType.DMA,
        pltpu.SemaphoreType.DMA,
        pltpu.SemaphoreType.DMA,
        pltpu.SemaphoreType.DMA,
        pltpu.SemaphoreType.DMA,
        pltpu.SemaphoreType.DMA,
    ],
)
def _sc_all(p1idx, p1vals, p2idx, p2vals, xz, zflat,
            acc, ibuf0, ibuf1, vbuf0, vbuf1, rbuf0, rbuf1, rbuf2,
            sidx0, sidx1, sidx2, sval0, sval1, sval2,
            isem0, isem1, vsem0, vsem1,
            gsem0, gsem1, gsem2, ssem0, ssem1, ssem2):
    cid = lax.axis_index("c")
    sid = lax.axis_index("s")
    ibufs = (ibuf0, ibuf1)
    vbufs = (vbuf0, vbuf1)
    rbufs = (rbuf0, rbuf1, rbuf2)
    sidxs = (sidx0, sidx1, sidx2)
    svals = (sval0, sval1, sval2)
    isems = (isem0, isem1)
    vsems = (vsem0, vsem1)
    gsems = (gsem0, gsem1, gsem2)
    ssems = (ssem0, ssem1, ssem2)

    def spmm(packed, pvals, x_hbm, chunk_base, z_row_base):
        _do_spmm(packed, pvals, x_hbm, zflat, chunk_base, z_row_base,
                 acc, ibufs, vbufs, rbufs, sidxs, svals,
                 isems, vsems, gsems, ssems, sid)

    def l1_body(p, _):
        use = cid * 2 + p
        z_slot = 2 * p + cid
        spmm(p1idx, p1vals, xz, use * N_CHUNKS, z_slot * N_S)
        return 0

    lax.fori_loop(0, 2, l1_body, 0)

    def l2_body(p, _):
        use = cid * 2 + p
        z_slot = jnp.where(cid == 0, 5 + p, jnp.where(p == 0, 4, 7))
        spmm(p2idx, p2vals, zflat, use * N_CHUNKS, z_slot * N_S)
        return 0

    lax.fori_loop(0, 2, l2_body, 0)


def _pack_edges(rows, cols, vals, x_slot):
    pad = E_PAD - E_EDGES
    c = jnp.pad(cols.astype(jnp.int32) + x_slot * N_S,
                (0, pad)).reshape(N_CHUNKS, 1, CHUNK)
    r = jnp.pad(rows.astype(jnp.int32), (0, pad)).reshape(N_CHUNKS, 1, CHUNK)
    v = jnp.pad(vals, (0, pad)).reshape(N_CHUNKS, CHUNK)
    return jnp.concatenate([c, r], axis=1), v


GB_ROWS = B // 32


@functools.partial(
    pl.kernel,
    mesh=_MESH,
    out_type=[jax.ShapeDtypeStruct((B, DIM), jnp.float32)] * 6,
    scratch_types=[
        pltpu.VMEM((GB_ROWS,), jnp.int32),
        pltpu.VMEM((GB_ROWS, DIM), jnp.float32),
        pltpu.SemaphoreType.DMA,
    ],
)
def _sc_gather(g_s, e_s, g_h, e_h, sids, hids, pos, neg,
               gs_sel, es_sel, gh_sel, eh_sel, ehp, ehn,
               idxv, robuf, sem):
    cid = lax.axis_index("c")
    sid = lax.axis_index("s")
    base = (sid * 2 + cid) * GB_ROWS

    def gather(src, idx_hbm, out):
        pltpu.sync_copy(idx_hbm.at[pl.ds(base, GB_ROWS)], idxv)
        pltpu.async_copy(src.at[idxv], robuf, sem).wait()
        pltpu.sync_copy(robuf, out.at[pl.ds(base, GB_ROWS)])

    gather(g_s, sids, gs_sel)
    gather(e_s, sids, es_sel)
    gather(g_h, hids, gh_sel)
    gather(e_h, hids, eh_sel)
    gather(e_h, pos, ehp)
    gather(e_h, neg, ehn)


MB = 1024
N_PAD = 10240


def _w_kernel(a_ref, b1_ref, b2_ref, o_ref):
    @pl.when(pl.program_id(0) == 0)
    def _():
        o_ref[...] = jnp.zeros_like(o_ref)

    o_ref[...] += jnp.dot(a_ref[...], b1_ref[...] + b2_ref[...],
                          preferred_element_type=jnp.float32)


def _lowrank_w(a, b1, b2):
    return pl.pallas_call(
        _w_kernel,
        grid=(N_PAD // MB,),
        in_specs=[
            pl.BlockSpec((RANK, MB), lambda k: (0, k)),
            pl.BlockSpec((MB, DIM), lambda k: (k, 0)),
            pl.BlockSpec((MB, DIM), lambda k: (k, 0)),
        ],
        out_specs=pl.BlockSpec((RANK, DIM), lambda k: (0, 0)),
        out_shape=jax.ShapeDtypeStruct((RANK, DIM), jnp.float32),
    )(a, b1, b2)


def _sums_kernel(es0, zs1, zs2, ess0, zss1, zss2,
                 eh0, zh1, zh2, ehh0, zhh1, zhh2,
                 ums, vms, ws, wh,
                 g_s, g_h, e_s, e_h, a_out, b_out, sqn):
    @pl.when(pl.program_id(0) == 0)
    def _():
        sqn[0, 0] = 0.0

    e_s[...] = es0[...] + zs1[...] + zs2[...]
    e_h[...] = eh0[...] + zh1[...] + zh2[...]
    a_out[...] = e_s[...] + ess0[...] + zss1[...] + zss2[...]
    b_out[...] = e_h[...] + ehh0[...] + zhh1[...] + zhh2[...]
    g_s[...] = es0[...] + jnp.dot(ums[...], ws[...],
                                  preferred_element_type=jnp.float32)
    g_h[...] = eh0[...] + jnp.dot(vms[...], wh[...],
                                  preferred_element_type=jnp.float32)
    sqn[0, 0] += (jnp.sum(es0[...] ** 2) + jnp.sum(eh0[...] ** 2)
                  + jnp.sum(ess0[...] ** 2) + jnp.sum(ehh0[...] ** 2))


def _sums_lowrank(es0, zs1, zs2, ess0, zss1, zss2,
                  eh0, zh1, zh2, ehh0, zhh1, zhh2, ums, vms, ws, wh):
    row_spec = pl.BlockSpec((MB, DIM), lambda k: (k, 0))
    rank_spec = pl.BlockSpec((MB, RANK), lambda k: (k, 0))
    w_spec = pl.BlockSpec((RANK, DIM), lambda k: (0, 0))
    shape = jax.ShapeDtypeStruct((N_PAD, DIM), jnp.float32)
    return pl.pallas_call(
        _sums_kernel,
        grid=(N_PAD // MB,),
        in_specs=[row_spec] * 12 + [rank_spec, rank_spec, w_spec, w_spec],
        out_specs=[row_spec] * 6 + [
            pl.BlockSpec(memory_space=pltpu.SMEM)],
        out_shape=[shape] * 6 + [
            jax.ShapeDtypeStruct((1, 1), jnp.float32)],
    )(es0, zs1, zs2, ess0, zss1, zss2, eh0, zh1, zh2, ehh0, zhh1, zhh2,
      ums, vms, ws, wh)


def _esynd_kernel(ps_ref, a_ref, g8_ref, b8_ref, e_ref, psum):
    k = pl.program_id(0)

    @pl.when(k == 0)
    def _():
        e_ref[...] = jnp.zeros_like(e_ref)
        psum[...] = jnp.zeros_like(psum)

    e_ref[...] += jnp.dot(ps_ref[...], a_ref[...],
                          preferred_element_type=jnp.float32)
    psum[...] += jnp.sum(ps_ref[...], axis=1, keepdims=True)

    @pl.when(k == pl.num_programs(0) - 1)
    def _():
        e = e_ref[...] / psum[...]
        mean = jnp.mean(e, axis=0, keepdims=True)
        var = jnp.mean(e * e, axis=0, keepdims=True) - mean * mean
        e = (e - mean) / jnp.sqrt(var + BN_EPS)
        e = e * g8_ref[0:1, :] + b8_ref[0:1, :]
        e_ref[...] = jnp.maximum(e, 0.0)


def _esynd_bn(ps_pad, a_pad, gamma8, beta8):
    return pl.pallas_call(
        _esynd_kernel,
        grid=(N_PAD // MB,),
        in_specs=[
            pl.BlockSpec((BU, MB), lambda k: (0, k)),
            pl.BlockSpec((MB, DIM), lambda k: (k, 0)),
            pl.BlockSpec((8, DIM), lambda k: (0, 0)),
            pl.BlockSpec((8, DIM), lambda k: (0, 0)),
        ],
        out_specs=pl.BlockSpec((BU, DIM), lambda k: (0, 0)),
        out_shape=jax.ShapeDtypeStruct((BU, DIM), jnp.float32),
        scratch_shapes=[pltpu.VMEM((BU, 1), jnp.float32)],
    )(ps_pad, a_pad, gamma8, beta8)


def _pre_kernel(e_ref, b_ref, o_ref):
    o_ref[...] = jax.lax.dot_general(
        e_ref[...], b_ref[...], (((1,), (1,)), ((), ())),
        preferred_element_type=jnp.float32)


def _pre_matmul(e_bn, b_pad):
    return pl.pallas_call(
        _pre_kernel,
        grid=(N_PAD // MB,),
        in_specs=[
            pl.BlockSpec((BU, DIM), lambda k: (0, 0)),
            pl.BlockSpec((MB, DIM), lambda k: (k, 0)),
        ],
        out_specs=pl.BlockSpec((BU, MB), lambda k: (0, k)),
        out_shape=jax.ShapeDtypeStruct((BU, N_PAD), jnp.float32),
    )(e_bn, b_pad)


def _scores_kernel(gs_ref, es_sel_ref, esb_ref, gh_ref, eh_sel_ref, ehb_ref,
                   ehp_ref, ehn_ref,
                   dens_ref, denh_ref, digs_ref, digh_ref, bpp_ref, bpn_ref):
    k = pl.program_id(0)

    @pl.when(k == 0)
    def _():
        dens_ref[...] = jnp.zeros_like(dens_ref)
        denh_ref[...] = jnp.zeros_like(denh_ref)
        digs_ref[...] = jnp.sum(gs_ref[...] * es_sel_ref[...], axis=1,
                                keepdims=True)
        digh_ref[...] = jnp.sum(gh_ref[...] * eh_sel_ref[...], axis=1,
                                keepdims=True)
        bpp_ref[...] = jnp.sum(es_sel_ref[...] * ehp_ref[...], axis=1,
                               keepdims=True)
        bpn_ref[...] = jnp.sum(es_sel_ref[...] * ehn_ref[...], axis=1,
                               keepdims=True)

    ss = jax.lax.dot_general(gs_ref[...], esb_ref[...],
                             (((1,), (1,)), ((), ())),
                             preferred_element_type=jnp.float32)
    dens_ref[...] += jnp.sum(jnp.exp(ss / TEMP), axis=1, keepdims=True)
    sh = jax.lax.dot_general(gh_ref[...], ehb_ref[...],
                             (((1,), (1,)), ((), ())),
                             preferred_element_type=jnp.float32)
    denh_ref[...] += jnp.sum(jnp.exp(sh / TEMP), axis=1, keepdims=True)

    @pl.when(k == pl.num_programs(0) - 1)
    def _():
        dens_ref[...] -= jnp.float32(N_PAD - N_S)
        denh_ref[...] -= jnp.float32(N_PAD - N_S)


def _scores(gs_sel, es_sel, es_pad, gh_sel, eh_sel, eh_pad, ehp, ehn):
    vec = jax.ShapeDtypeStruct((B, 1), jnp.float32)
    sel_spec = pl.BlockSpec((B, DIM), lambda k: (0, 0))
    blk_spec = pl.BlockSpec((MB, DIM), lambda k: (k, 0))
    out_spec = pl.BlockSpec((B, 1), lambda k: (0, 0))
    return pl.pallas_call(
        _scores_kernel,
        grid=(N_PAD // MB,),
        in_specs=[sel_spec, sel_spec, blk_spec, sel_spec, sel_spec,
                  blk_spec, sel_spec, sel_spec],
        out_specs=[out_spec] * 6,
        out_shape=[vec] * 6,
    )(gs_sel, es_sel, es_pad, gh_sel, eh_sel, eh_pad, ehp, ehn)


def _final_kernel(dens_ref, denh_ref, digs_ref, digh_ref, bpp_ref, bpn_ref,
                  sqn8_ref, g8_ref, b8_ref,
                  loss_ref, lossr_ref, losss_ref):
    neg = (jnp.mean(jnp.log(dens_ref[...] + 1e-08))
           + jnp.mean(jnp.log(denh_ref[...] + 1e-08)))
    pos = (jnp.mean(jnp.clip(digs_ref[...] / TEMP, -5.0, 5.0))
           + jnp.mean(jnp.clip(digh_ref[...] / TEMP, -5.0, 5.0)))
    loss_s = LAMBDA_1 * (neg - pos)
    diff = bpp_ref[...] - bpn_ref[...]
    loss_r = -jnp.mean(jnp.log(jax.nn.sigmoid(diff)))
    reg = (sqn8_ref[0, 0] + jnp.sum(g8_ref[0:1, :] ** 2)
           + jnp.sum(b8_ref[0:1, :] ** 2)) * LAMBDA_2
    loss_ref[0, 0] = loss_r + loss_s + reg
    lossr_ref[0, 0] = loss_r
    losss_ref[0, 0] = loss_s


def _finalize(dens, denh, digs, digh, bpp, bpn, sqn8, gamma8, beta8):
    vec_spec = pl.BlockSpec((B, 1), lambda: (0, 0))
    row_spec = pl.BlockSpec((8, DIM), lambda: (0, 0))
    scal = jax.ShapeDtypeStruct((1, 1), jnp.float32)
    return pl.pallas_call(
        _final_kernel,
        grid=(),
        in_specs=[vec_spec] * 6 + [row_spec] * 3,
        out_specs=[pl.BlockSpec(memory_space=pltpu.SMEM)] * 3,
        out_shape=[scal] * 3,
    )(dens, denh, digs, digh, bpp, bpn, sqn8, gamma8, beta8)


def kernel(sids, hids, pos, neg, ps, E_s_0, E_h_0, E_ss_0, E_hh_0,
           adj_rows, adj_cols, adj_vals, ss_rows, ss_cols, ss_vals,
           hh_rows, hh_cols, hh_vals, u_mul_s, vt, v_mul_s, ut,
           bn_gamma, bn_beta):
    xz = jnp.concatenate([E_s_0, E_h_0, E_ss_0, E_hh_0], axis=0)
    l1_uses = [
        _pack_edges(adj_rows, adj_cols, adj_vals, 1),
        _pack_edges(ss_rows, ss_cols, ss_vals, 2),
        _pack_edges(adj_cols, adj_rows, adj_vals, 0),
        _pack_edges(hh_rows, hh_cols, hh_vals, 3),
    ]
    l2_uses = [
        _pack_edges(adj_cols, adj_rows, adj_vals, 0),
        _pack_edges(ss_rows, ss_cols, ss_vals, 2),
        _pack_edges(adj_rows, adj_cols, adj_vals, 1),
        _pack_edges(hh_rows, hh_cols, hh_vals, 3),
    ]
    p1idx = jnp.concatenate([u[0] for u in l1_uses], axis=0)
    p1vals = jnp.concatenate([u[1] for u in l1_uses], axis=0)
    p2idx = jnp.concatenate([u[0] for u in l2_uses], axis=0)
    p2vals = jnp.concatenate([u[1] for u in l2_uses], axis=0)

    zflat = _sc_all(p1idx, p1vals, p2idx, p2vals, xz)

    def padn(x):
        return jnp.pad(x, ((0, N_PAD - N_S), (0, 0)))

    def zslot(k):
        return padn(zflat[k * N_S:(k + 1) * N_S])

    zs1, zh1, zss1, zhh1, zs2, zh2, zss2, zhh2 = (zslot(k) for k in range(8))
    es0, eh0, ess0, ehh0 = padn(E_s_0), padn(E_h_0), padn(E_ss_0), padn(E_hh_0)

    vt_pad = jnp.pad(vt, ((0, 0), (0, N_PAD - N_S)))
    ut_pad = jnp.pad(ut, ((0, 0), (0, N_PAD - N_S)))
    w_s = _lowrank_w(vt_pad, eh0, zh1)
    w_h = _lowrank_w(ut_pad, es0, zs1)

    ums = jnp.pad(u_mul_s, ((0, N_PAD - N_S), (0, 0)))
    vms = jnp.pad(v_mul_s, ((0, N_PAD - N_S), (0, 0)))
    g_s, g_h, e_s, e_h, a_sum, b_sum, sqn = _sums_lowrank(
        es0, zs1, zs2, ess0, zss1, zss2,
        eh0, zh1, zh2, ehh0, zhh1, zhh2, ums, vms, w_s, w_h)

    gamma8 = jnp.broadcast_to(bn_gamma[None, :], (8, DIM))
    beta8 = jnp.broadcast_to(bn_beta[None, :], (8, DIM))
    ps_pad = jnp.pad(ps, ((0, 0), (0, N_PAD - N_S)))
    e_bn = _esynd_bn(ps_pad, a_sum, gamma8, beta8)
    pre = _pre_matmul(e_bn, b_sum)[:, :N_S]

    gs_sel, es_sel, gh_sel, eh_sel, ehp, ehn = _sc_gather(
        g_s, e_s, g_h, e_h, sids.astype(jnp.int32), hids.astype(jnp.int32),
        pos.astype(jnp.int32), neg.astype(jnp.int32))

    dens, denh, digs, digh, bpp, bpn = _scores(
        gs_sel, es_sel, e_s, gh_sel, eh_sel, e_h, ehp, ehn)

    sqn8 = jnp.broadcast_to(sqn, (8, DIM))
    loss, loss_r, loss_s = _finalize(dens, denh, digs, digh, bpp, bpn,
                                     sqn8, gamma8, beta8)
    return (loss.reshape(()), loss_r.reshape(()), loss_s.reshape(()), pre)


# speedup vs baseline: 1.0320x; 1.0320x over previous
import functools

import jax
import jax.numpy as jnp
from jax import lax
from jax.experimental import pallas as pl
from jax.experimental.pallas import tpu as pltpu
from jax.experimental.pallas import tpu_sc as plsc

N_S = 10000
N_H = 10000
DIM = 128
RANK = 64
LAYER = 2
TEMP = 0.2
LAMBDA_1 = 0.2
LAMBDA_2 = 1e-07
BN_EPS = 1e-05
B = 4096
BU = 1024

E_EDGES = 320000
CHUNK = 112
CHUNKS_PER_TILE = 180
N_CHUNKS = CHUNKS_PER_TILE * 16
E_PAD = N_CHUNKS * CHUNK
ROWS_MAIN = 624
ROWS_LAST = N_S - 15 * ROWS_MAIN

N_PAD = 10240
MB = 1024
SPB = N_PAD // MB

K_PAD = 10240
K_TILE = 1024

_MESH = plsc.VectorSubcoreMesh(core_axis_name="c", subcore_axis_name="s")


def _fill_zeros(rbuf):
    zeros16 = jnp.zeros((16,), jnp.float32)

    def body(r, _):
        for d in range(8):
            rbuf[r, pl.ds(d * 16, 16)] = zeros16
        return 0

    lax.fori_loop(0, CHUNK, body, 0)


def _do_spmm(packed, pvals, x_hbm, z_hbm, chunk_base, z_row_base,
             acc, ibufs, vbufs, rbufs, sidxs, svals,
             isems, vsems, gsems, ssems, sid):
    base_row = sid * ROWS_MAIN
    rbuf0 = rbufs[0]

    _fill_zeros(rbuf0)

    @pl.when(sid < 15)
    def _():
        for k in range(5):
            pltpu.sync_copy(rbuf0, acc.at[pl.ds(base_row + k * CHUNK, CHUNK)])
        rem = ROWS_MAIN - 5 * CHUNK
        pltpu.sync_copy(rbuf0.at[pl.ds(0, rem)],
                        acc.at[pl.ds(base_row + 5 * CHUNK, rem)])

    @pl.when(sid == 15)
    def _():
        base = 15 * ROWS_MAIN
        for k in range(5):
            pltpu.sync_copy(rbuf0, acc.at[pl.ds(base + k * CHUNK, CHUNK)])
        rem = ROWS_LAST - 5 * CHUNK
        pltpu.sync_copy(rbuf0.at[pl.ds(0, rem)],
                        acc.at[pl.ds(base + 5 * CHUNK, rem)])

    @pl.when(sid < 2)
    def _():
        pltpu.sync_copy(rbuf0.at[pl.ds(0, 120)],
                        z_hbm.at[pl.ds(z_row_base + N_S + sid * 120, 120)])

    plsc.subcore_barrier()

    base_chunk = chunk_base + sid * CHUNKS_PER_TILE
    n = CHUNKS_PER_TILE

    def idx_start(j, p2):
        pltpu.async_copy(packed.at[base_chunk + j], ibufs[p2], isems[p2])
        pltpu.async_copy(pvals.at[base_chunk + j], vbufs[p2], vsems[p2])

    def idx_wait(j, p2):
        pltpu.make_async_copy(packed.at[base_chunk + j], ibufs[p2],
                              isems[p2]).wait()
        pltpu.make_async_copy(pvals.at[base_chunk + j], vbufs[p2],
                              vsems[p2]).wait()

    def gather_start(j, p2, p3):
        pltpu.async_copy(x_hbm.at[ibufs[p2].at[0]], rbufs[p3], gsems[p3])

    def gather_wait(p2, p3):
        pltpu.make_async_copy(x_hbm.at[ibufs[p2].at[0]], rbufs[p3],
                              gsems[p3]).wait()

    def scatter_start(p3):
        pltpu.async_copy(rbufs[p3], acc.at[sidxs[p3]], ssems[p3], add=True)

    def scatter_wait(p3):
        pltpu.make_async_copy(rbufs[p3], acc.at[sidxs[p3]],
                              ssems[p3]).wait()

    def half(j, k):
        p3 = k % 3
        p2 = k % 2
        rbuf, sidx, sval = rbufs[p3], sidxs[p3], svals[p3]
        ibuf, vbuf = ibufs[p2], vbufs[p2]

        @pl.when(j + 1 < n)
        def _():
            idx_wait(j + 1, (k + 1) % 2)

        @pl.when(j >= 2)
        def _():
            scatter_wait((k + 1) % 3)

        @pl.when(j + 1 < n)
        def _():
            gather_start(j + 1, (k + 1) % 2, (k + 1) % 3)

        gather_wait(p2, p3)
        for g in range(CHUNK // 16):
            sl = pl.ds(g * 16, 16)
            sidx[sl] = ibuf[1, sl]
            sval[sl] = vbuf[sl]

        @pl.when(j + 2 < n)
        def _():
            idx_start(j + 2, k % 2)

        def scale_body(g, _):
            vv = sval[pl.ds(g * 16, 16)]
            for l in range(16):
                v = vv[l]
                e = g * 16 + l
                for d in range(8):
                    sl = pl.ds(d * 16, 16)
                    rbuf[e, sl] = rbuf[e, sl] * v
            return 0

        lax.fori_loop(0, CHUNK // 16, scale_body, 0)
        scatter_start(p3)

    idx_start(0, 0)
    idx_wait(0, 0)
    gather_start(0, 0, 0)
    idx_start(1, 1)

    def chunk_body(i6, _):
        for k in range(6):
            half(6 * i6 + k, k)
        return 0

    lax.fori_loop(0, n // 6, chunk_body, 0)
    scatter_wait((n - 2) % 3)
    scatter_wait((n - 1) % 3)
    plsc.subcore_barrier()

    zb = z_row_base + base_row

    @pl.when(sid < 15)
    def _():
        for k in range(5):
            pltpu.sync_copy(acc.at[pl.ds(base_row + k * CHUNK, CHUNK)],
                            z_hbm.at[pl.ds(zb + k * CHUNK, CHUNK)])
        rem = ROWS_MAIN - 5 * CHUNK
        pltpu.sync_copy(acc.at[pl.ds(base_row + 5 * CHUNK, rem)],
                        z_hbm.at[pl.ds(zb + 5 * CHUNK, rem)])

    @pl.when(sid == 15)
    def _():
        base = 15 * ROWS_MAIN
        zbase = z_row_base + base
        for k in range(5):
            pltpu.sync_copy(acc.at[pl.ds(base + k * CHUNK, CHUNK)],
                            z_hbm.at[pl.ds(zbase + k * CHUNK, CHUNK)])
        rem = ROWS_LAST - 5 * CHUNK
        pltpu.sync_copy(acc.at[pl.ds(base + 5 * CHUNK, rem)],
                        z_hbm.at[pl.ds(zbase + 5 * CHUNK, rem)])

    plsc.subcore_barrier()


@functools.partial(
    pl.kernel,
    mesh=_MESH,
    out_type=jax.ShapeDtypeStruct((8 * N_PAD, DIM), jnp.float32),
    scratch_types=[
        pltpu.VMEM_SHARED((N_S, DIM), jnp.float32),
        pltpu.VMEM((2, CHUNK), jnp.int32),
        pltpu.VMEM((2, CHUNK), jnp.int32),
        pltpu.VMEM((CHUNK,), jnp.float32),
        pltpu.VMEM((CHUNK,), jnp.float32),
        pltpu.VMEM((CHUNK, DIM), jnp.float32),
        pltpu.VMEM((CHUNK, DIM), jnp.float32),
        pltpu.VMEM((CHUNK, DIM), jnp.float32),
        pltpu.VMEM((CHUNK,), jnp.int32),
        pltpu.VMEM((CHUNK,), jnp.int32),
        pltpu.VMEM((CHUNK,), jnp.int32),
        pltpu.VMEM((CHUNK,), jnp.float32),
        pltpu.VMEM((CHUNK,), jnp.float32),
        pltpu.VMEM((CHUNK,), jnp.float32),
        pltpu.SemaphoreType.DMA,
        pltpu.SemaphoreType.DMA,
        pltpu.SemaphoreType.DMA,
        pltpu.SemaphoreType.DMA,
        pltpu.SemaphoreType.DMA,
        pltpu.SemaphoreType.DMA,
        pltpu.SemaphoreType.DMA,
        pltpu.SemaphoreType.DMA,
        pltpu.SemaphoreType.DMA,
        pltpu.SemaphoreType.DMA,
    ],
)
def _sc_all(p1idx, p1vals, p2idx, p2vals, xz, zflat,
            acc, ibuf0, ibuf1, vbuf0, vbuf1, rbuf0, rbuf1, rbuf2,
            sidx0, sidx1, sidx2, sval0, sval1, sval2,
            isem0, isem1, vsem0, vsem1,
            gsem0, gsem1, gsem2, ssem0, ssem1, ssem2):
    cid = lax.axis_index("c")
    sid = lax.axis_index("s")
    ibufs = (ibuf0, ibuf1)
    vbufs = (vbuf0, vbuf1)
    rbufs = (rbuf0, rbuf1, rbuf2)
    sidxs = (sidx0, sidx1, sidx2)
    svals = (sval0, sval1, sval2)
    isems = (isem0, isem1)
    vsems = (vsem0, vsem1)
    gsems = (gsem0, gsem1, gsem2)
    ssems = (ssem0, ssem1, ssem2)

    def spmm(packed, pvals, x_hbm, chunk_base, z_row_base):
        _do_spmm(packed, pvals, x_hbm, zflat, chunk_base, z_row_base,
                 acc, ibufs, vbufs, rbufs, sidxs, svals,
                 isems, vsems, gsems, ssems, sid)

    def l1_body(p, _):
        use = cid * 2 + p
        z_slot = 2 * p + cid
        spmm(p1idx, p1vals, xz, use * N_CHUNKS, z_slot * N_PAD)
        return 0

    lax.fori_loop(0, 2, l1_body, 0)

    def l2_body(p, _):
        use = cid * 2 + p
        z_slot = jnp.where(cid == 0, 5 + p, jnp.where(p == 0, 4, 7))
        spmm(p2idx, p2vals, zflat, use * N_CHUNKS, z_slot * N_PAD)
        return 0

    lax.fori_loop(0, 2, l2_body, 0)


def _pack_edges(rows, cols, vals, x_slot):
    pad = E_PAD - E_EDGES
    c = jnp.pad(cols.astype(jnp.int32) + x_slot * N_PAD,
                (0, pad)).reshape(N_CHUNKS, 1, CHUNK)
    r = jnp.pad(rows.astype(jnp.int32), (0, pad)).reshape(N_CHUNKS, 1, CHUNK)
    v = jnp.pad(vals, (0, pad)).reshape(N_CHUNKS, CHUNK)
    return jnp.concatenate([c, r], axis=1), v


GB_ROWS = B // 32


@functools.partial(
    pl.kernel,
    mesh=_MESH,
    out_type=[jax.ShapeDtypeStruct((B, DIM), jnp.float32)] * 6,
    scratch_types=[
        pltpu.VMEM((GB_ROWS,), jnp.int32),
        pltpu.VMEM((GB_ROWS, DIM), jnp.float32),
        pltpu.SemaphoreType.DMA,
    ],
)
def _sc_gather(g_s, e_s, g_h, e_h, sids, hids, pos, neg,
               gs_sel, es_sel, gh_sel, eh_sel, ehp, ehn,
               idxv, robuf, sem):
    cid = lax.axis_index("c")
    sid = lax.axis_index("s")
    base = (sid * 2 + cid) * GB_ROWS

    def gather(src, idx_hbm, out):
        pltpu.sync_copy(idx_hbm.at[pl.ds(base, GB_ROWS)], idxv)
        pltpu.async_copy(src.at[idxv], robuf, sem).wait()
        pltpu.sync_copy(robuf, out.at[pl.ds(base, GB_ROWS)])

    gather(g_s, sids, gs_sel)
    gather(e_s, sids, es_sel)
    gather(g_h, hids, gh_sel)
    gather(e_h, hids, eh_sel)
    gather(e_h, pos, ehp)
    gather(e_h, neg, ehn)


def _slot_spec(slot):
    return pl.BlockSpec((MB, DIM), lambda k, s=slot: (k + s * SPB, 0))


def _w2_kernel(vt_ref, ut_ref, eh0_ref, zh1_ref, es0_ref, zs1_ref,
               ws_ref, wh_ref):
    @pl.when(pl.program_id(0) == 0)
    def _():
        ws_ref[...] = jnp.zeros_like(ws_ref)
        wh_ref[...] = jnp.zeros_like(wh_ref)

    ws_ref[...] += jnp.dot(vt_ref[...], eh0_ref[...] + zh1_ref[...],
                           preferred_element_type=jnp.float32)
    wh_ref[...] += jnp.dot(ut_ref[...], es0_ref[...] + zs1_ref[...],
                           preferred_element_type=jnp.float32)


def _lowrank_w(vt_pad, ut_pad, xz, zflat):
    w_spec = pl.BlockSpec((RANK, DIM), lambda k: (0, 0))
    return pl.pallas_call(
        _w2_kernel,
        grid=(SPB,),
        in_specs=[
            pl.BlockSpec((RANK, MB), lambda k: (0, k)),
            pl.BlockSpec((RANK, MB), lambda k: (0, k)),
            _slot_spec(1),
            _slot_spec(1),
            _slot_spec(0),
            _slot_spec(0),
        ],
        out_specs=[w_spec, w_spec],
        out_shape=[jax.ShapeDtypeStruct((RANK, DIM), jnp.float32)] * 2,
    )(vt_pad, ut_pad, xz, zflat, xz, zflat)


def _sums_kernel(es0, zs1, zs2, ess0, zss1, zss2,
                 eh0, zh1, zh2, ehh0, zhh1, zhh2,
                 ums, vms, ws, wh,
                 g_s, g_h, e_s, e_h, a_out, b_out, sqn):
    @pl.when(pl.program_id(0) == 0)
    def _():
        sqn[0, 0] = 0.0

    e_s[...] = es0[...] + zs1[...] + zs2[...]
    e_h[...] = eh0[...] + zh1[...] + zh2[...]
    a_out[...] = e_s[...] + ess0[...] + zss1[...] + zss2[...]
    b_out[...] = e_h[...] + ehh0[...] + zhh1[...] + zhh2[...]
    g_s[...] = es0[...] + jnp.dot(ums[...], ws[...],
                                  preferred_element_type=jnp.float32)
    g_h[...] = eh0[...] + jnp.dot(vms[...], wh[...],
                                  preferred_element_type=jnp.float32)
    sqn[0, 0] += (jnp.sum(es0[...] ** 2) + jnp.sum(eh0[...] ** 2)
                  + jnp.sum(ess0[...] ** 2) + jnp.sum(ehh0[...] ** 2))


def _sums_lowrank(xz, zflat, ums, vms, ws, wh):
    row_spec = pl.BlockSpec((MB, DIM), lambda k: (k, 0))
    rank_spec = pl.BlockSpec((MB, RANK), lambda k: (k, 0))
    w_spec = pl.BlockSpec((RANK, DIM), lambda k: (0, 0))
    shape = jax.ShapeDtypeStruct((N_PAD, DIM), jnp.float32)
    in_specs = [
        _slot_spec(0), _slot_spec(0), _slot_spec(4),
        _slot_spec(2), _slot_spec(2), _slot_spec(6),
        _slot_spec(1), _slot_spec(1), _slot_spec(5),
        _slot_spec(3), _slot_spec(3), _slot_spec(7),
        rank_spec, rank_spec, w_spec, w_spec,
    ]
    return pl.pallas_call(
        _sums_kernel,
        grid=(SPB,),
        in_specs=in_specs,
        out_specs=[row_spec] * 6 + [
            pl.BlockSpec(memory_space=pltpu.SMEM)],
        out_shape=[shape] * 6 + [
            jax.ShapeDtypeStruct((1, 1), jnp.float32)],
    )(xz, zflat, zflat, xz, zflat, zflat, xz, zflat, zflat,
      xz, zflat, zflat, ums, vms, ws, wh)


def _esynd_kernel(ps_ref, a_ref, g8_ref, b8_ref, e_ref, psum):
    k = pl.program_id(0)

    @pl.when(k == 0)
    def _():
        e_ref[...] = jnp.zeros_like(e_ref)
        psum[...] = jnp.zeros_like(psum)

    e_ref[...] += jnp.dot(ps_ref[...], a_ref[...],
                          preferred_element_type=jnp.float32)
    psum[...] += jnp.sum(ps_ref[...], axis=1, keepdims=True)

    @pl.when(k == pl.num_programs(0) - 1)
    def _():
        e = e_ref[...] / psum[...]
        mean = jnp.mean(e, axis=0, keepdims=True)
        var = jnp.mean(e * e, axis=0, keepdims=True) - mean * mean
        e = (e - mean) / jnp.sqrt(var + BN_EPS)
        e = e * g8_ref[0:1, :] + b8_ref[0:1, :]
        e_ref[...] = jnp.maximum(e, 0.0)


def _esynd_bn(ps_pad, a_pad, gamma8, beta8):
    return pl.pallas_call(
        _esynd_kernel,
        grid=(N_PAD // MB,),
        in_specs=[
            pl.BlockSpec((BU, MB), lambda k: (0, k)),
            pl.BlockSpec((MB, DIM), lambda k: (k, 0)),
            pl.BlockSpec((8, DIM), lambda k: (0, 0)),
            pl.BlockSpec((8, DIM), lambda k: (0, 0)),
        ],
        out_specs=pl.BlockSpec((BU, DIM), lambda k: (0, 0)),
        out_shape=jax.ShapeDtypeStruct((BU, DIM), jnp.float32),
        scratch_shapes=[pltpu.VMEM((BU, 1), jnp.float32)],
    )(ps_pad, a_pad, gamma8, beta8)


def _pre_kernel(e_ref, b_ref, o_ref):
    o_ref[...] = jax.lax.dot_general(
        e_ref[...], b_ref[...], (((1,), (1,)), ((), ())),
        preferred_element_type=jnp.float32)


def _pre_matmul(e_bn, b_pad):
    return pl.pallas_call(
        _pre_kernel,
        grid=(N_PAD // MB,),
        in_specs=[
            pl.BlockSpec((BU, DIM), lambda k: (0, 0)),
            pl.BlockSpec((MB, DIM), lambda k: (k, 0)),
        ],
        out_specs=pl.BlockSpec((BU, MB), lambda k: (0, k)),
        out_shape=jax.ShapeDtypeStruct((BU, N_PAD), jnp.float32),
    )(e_bn, b_pad)


def _scores_kernel(gs_ref, es_sel_ref, esb_ref, gh_ref, eh_sel_ref, ehb_ref,
                   ehp_ref, ehn_ref,
                   dens_ref, denh_ref, digs_ref, digh_ref, bpp_ref, bpn_ref):
    k = pl.program_id(0)

    @pl.when(k == 0)
    def _():
        dens_ref[...] = jnp.zeros_like(dens_ref)
        denh_ref[...] = jnp.zeros_like(denh_ref)
        digs_ref[...] = jnp.sum(gs_ref[...] * es_sel_ref[...], axis=1,
                                keepdims=True)
        digh_ref[...] = jnp.sum(gh_ref[...] * eh_sel_ref[...], axis=1,
                                keepdims=True)
        bpp_ref[...] = jnp.sum(es_sel_ref[...] * ehp_ref[...], axis=1,
                               keepdims=True)
        bpn_ref[...] = jnp.sum(es_sel_ref[...] * ehn_ref[...], axis=1,
                               keepdims=True)

    ss = jax.lax.dot_general(gs_ref[...], esb_ref[...],
                             (((1,), (1,)), ((), ())),
                             preferred_element_type=jnp.float32)
    dens_ref[...] += jnp.sum(jnp.exp(ss / TEMP), axis=1, keepdims=True)
    sh = jax.lax.dot_general(gh_ref[...], ehb_ref[...],
                             (((1,), (1,)), ((), ())),
                             preferred_element_type=jnp.float32)
    denh_ref[...] += jnp.sum(jnp.exp(sh / TEMP), axis=1, keepdims=True)

    @pl.when(k == pl.num_programs(0) - 1)
    def _():
        dens_ref[...] -= jnp.float32(N_PAD - N_S)
        denh_ref[...] -= jnp.float32(N_PAD - N_S)


def _scores(gs_sel, es_sel, es_pad, gh_sel, eh_sel, eh_pad, ehp, ehn):
    vec = jax.ShapeDtypeStruct((B, 1), jnp.float32)
    sel_spec = pl.BlockSpec((B, DIM), lambda k: (0, 0))
    blk_spec = pl.BlockSpec((MB, DIM), lambda k: (k, 0))
    out_spec = pl.BlockSpec((B, 1), lambda k: (0, 0))
    return pl.pallas_call(
        _scores_kernel,
        grid=(N_PAD // MB,),
        in_specs=[sel_spec, sel_spec, blk_spec, sel_spec, sel_spec,
                  blk_spec, sel_spec, sel_spec],
        out_specs=[out_spec] * 6,
        out_shape=[vec] * 6,
    )(gs_sel, es_sel, es_pad, gh_sel, eh_sel, eh_pad, ehp, ehn)


def _final_kernel(dens_ref, denh_ref, digs_ref, digh_ref, bpp_ref, bpn_ref,
                  sqn8_ref, g8_ref, b8_ref,
                  loss_ref, lossr_ref, losss_ref):
    neg = (jnp.mean(jnp.log(dens_ref[...] + 1e-08))
           + jnp.mean(jnp.log(denh_ref[...] + 1e-08)))
    pos = (jnp.mean(jnp.clip(digs_ref[...] / TEMP, -5.0, 5.0))
           + jnp.mean(jnp.clip(digh_ref[...] / TEMP, -5.0, 5.0)))
    loss_s = LAMBDA_1 * (neg - pos)
    diff = bpp_ref[...] - bpn_ref[...]
    loss_r = -jnp.mean(jnp.log(jax.nn.sigmoid(diff)))
    reg = (sqn8_ref[0, 0] + jnp.sum(g8_ref[0:1, :] ** 2)
           + jnp.sum(b8_ref[0:1, :] ** 2)) * LAMBDA_2
    loss_ref[0, 0] = loss_r + loss_s + reg
    lossr_ref[0, 0] = loss_r
    losss_ref[0, 0] = loss_s


def _finalize(dens, denh, digs, digh, bpp, bpn, sqn8, gamma8, beta8):
    vec_spec = pl.BlockSpec((B, 1), lambda: (0, 0))
    row_spec = pl.BlockSpec((8, DIM), lambda: (0, 0))
    scal = jax.ShapeDtypeStruct((1, 1), jnp.float32)
    return pl.pallas_call(
        _final_kernel,
        grid=(),
        in_specs=[vec_spec] * 6 + [row_spec] * 3,
        out_specs=[pl.BlockSpec(memory_space=pltpu.SMEM)] * 3,
        out_shape=[scal] * 3,
    )(dens, denh, digs, digh, bpp, bpn, sqn8, gamma8, beta8)


def kernel(sids, hids, pos, neg, ps, E_s_0, E_h_0, E_ss_0, E_hh_0,
           adj_rows, adj_cols, adj_vals, ss_rows, ss_cols, ss_vals,
           hh_rows, hh_cols, hh_vals, u_mul_s, vt, v_mul_s, ut,
           bn_gamma, bn_beta):
    zpad = jnp.zeros((N_PAD - N_S, DIM), jnp.float32)
    xz = jnp.concatenate([E_s_0, zpad, E_h_0, zpad, E_ss_0, zpad,
                          E_hh_0, zpad], axis=0)
    l1_uses = [
        _pack_edges(adj_rows, adj_cols, adj_vals, 1),
        _pack_edges(ss_rows, ss_cols, ss_vals, 2),
        _pack_edges(adj_cols, adj_rows, adj_vals, 0),
        _pack_edges(hh_rows, hh_cols, hh_vals, 3),
    ]
    l2_uses = [
        _pack_edges(adj_cols, adj_rows, adj_vals, 0),
        _pack_edges(ss_rows, ss_cols, ss_vals, 2),
        _pack_edges(adj_rows, adj_cols, adj_vals, 1),
        _pack_edges(hh_rows, hh_cols, hh_vals, 3),
    ]
    p1idx = jnp.concatenate([u[0] for u in l1_uses], axis=0)
    p1vals = jnp.concatenate([u[1] for u in l1_uses], axis=0)
    p2idx = jnp.concatenate([u[0] for u in l2_uses], axis=0)
    p2vals = jnp.concatenate([u[1] for u in l2_uses], axis=0)

    zflat = _sc_all(p1idx, p1vals, p2idx, p2vals, xz)

    vt_pad = jnp.pad(vt, ((0, 0), (0, N_PAD - N_S)))
    ut_pad = jnp.pad(ut, ((0, 0), (0, N_PAD - N_S)))
    w_s, w_h = _lowrank_w(vt_pad, ut_pad, xz, zflat)

    ums = jnp.pad(u_mul_s, ((0, N_PAD - N_S), (0, 0)))
    vms = jnp.pad(v_mul_s, ((0, N_PAD - N_S), (0, 0)))
    g_s, g_h, e_s, e_h, a_sum, b_sum, sqn = _sums_lowrank(
        xz, zflat, ums, vms, w_s, w_h)

    gamma8 = jnp.broadcast_to(bn_gamma[None, :], (8, DIM))
    beta8 = jnp.broadcast_to(bn_beta[None, :], (8, DIM))
    ps_pad = jnp.pad(ps, ((0, 0), (0, N_PAD - N_S)))
    e_bn = _esynd_bn(ps_pad, a_sum, gamma8, beta8)
    pre = _pre_matmul(e_bn, b_sum)[:, :N_S]

    gs_sel, es_sel, gh_sel, eh_sel, ehp, ehn = _sc_gather(
        g_s, e_s, g_h, e_h, sids.astype(jnp.int32), hids.astype(jnp.int32),
        pos.astype(jnp.int32), neg.astype(jnp.int32))

    dens, denh, digs, digh, bpp, bpn = _scores(
        gs_sel, es_sel, e_s, gh_sel, eh_sel, e_h, ehp, ehn)

    sqn8 = jnp.broadcast_to(sqn, (8, DIM))
    loss, loss_r, loss_s = _finalize(dens, denh, digs, digh, bpp, bpn,
                                     sqn8, gamma8, beta8)
    return (loss.reshape(()), loss_r.reshape(()), loss_s.reshape(()), pre)


# speedup vs baseline: 1.0891x; 1.0553x over previous
import functools

import jax
import jax.numpy as jnp
from jax import lax
from jax.experimental import pallas as pl
from jax.experimental.pallas import tpu as pltpu
from jax.experimental.pallas import tpu_sc as plsc

N_S = 10000
N_H = 10000
DIM = 128
RANK = 64
LAYER = 2
TEMP = 0.2
LAMBDA_1 = 0.2
LAMBDA_2 = 1e-07
BN_EPS = 1e-05
B = 4096
BU = 1024

E_EDGES = 320000
CHUNK = 112
CHUNKS_PER_TILE = 180
N_CHUNKS = CHUNKS_PER_TILE * 16
E_PAD = N_CHUNKS * CHUNK
ROWS_MAIN = 624
ROWS_LAST = N_S - 15 * ROWS_MAIN

N_PAD = 10240
MB = 1024
SPB = N_PAD // MB

K_PAD = 10240
K_TILE = 1024

_MESH = plsc.VectorSubcoreMesh(core_axis_name="c", subcore_axis_name="s")


def _fill_zeros(rbuf):
    zeros16 = jnp.zeros((16,), jnp.float32)

    def body(r, _):
        for d in range(8):
            rbuf[r, pl.ds(d * 16, 16)] = zeros16
        return 0

    lax.fori_loop(0, CHUNK, body, 0)


def _do_spmm(packed, pvals, x_hbm, z_hbm, chunk_base, z_row_base,
             acc, ibufs, vbufs, rbufs, sidxs, svals,
             isems, vsems, gsems, ssems, sid):
    base_row = sid * ROWS_MAIN
    rbuf0 = rbufs[0]

    _fill_zeros(rbuf0)

    @pl.when(sid < 15)
    def _():
        for k in range(5):
            pltpu.sync_copy(rbuf0, acc.at[pl.ds(base_row + k * CHUNK, CHUNK)])
        rem = ROWS_MAIN - 5 * CHUNK
        pltpu.sync_copy(rbuf0.at[pl.ds(0, rem)],
                        acc.at[pl.ds(base_row + 5 * CHUNK, rem)])

    @pl.when(sid == 15)
    def _():
        base = 15 * ROWS_MAIN
        for k in range(5):
            pltpu.sync_copy(rbuf0, acc.at[pl.ds(base + k * CHUNK, CHUNK)])
        rem = ROWS_LAST - 5 * CHUNK
        pltpu.sync_copy(rbuf0.at[pl.ds(0, rem)],
                        acc.at[pl.ds(base + 5 * CHUNK, rem)])

    @pl.when(sid < 2)
    def _():
        pltpu.sync_copy(rbuf0.at[pl.ds(0, 120)],
                        z_hbm.at[pl.ds(z_row_base + N_S + sid * 120, 120)])

    plsc.subcore_barrier()

    base_chunk = chunk_base + sid * CHUNKS_PER_TILE
    n = CHUNKS_PER_TILE

    def idx_start(j, p2):
        pltpu.async_copy(packed.at[base_chunk + j], ibufs[p2], isems[p2])
        pltpu.async_copy(pvals.at[base_chunk + j], vbufs[p2], vsems[p2])

    def idx_wait(j, p2):
        pltpu.make_async_copy(packed.at[base_chunk + j], ibufs[p2],
                              isems[p2]).wait()
        pltpu.make_async_copy(pvals.at[base_chunk + j], vbufs[p2],
                              vsems[p2]).wait()

    def gather_start(j, p2, p3):
        pltpu.async_copy(x_hbm.at[ibufs[p2].at[0]], rbufs[p3], gsems[p3])

    def gather_wait(p2, p3):
        pltpu.make_async_copy(x_hbm.at[ibufs[p2].at[0]], rbufs[p3],
                              gsems[p3]).wait()

    def scatter_start(p3):
        pltpu.async_copy(rbufs[p3], acc.at[sidxs[p3]], ssems[p3], add=True)

    def scatter_wait(p3):
        pltpu.make_async_copy(rbufs[p3], acc.at[sidxs[p3]],
                              ssems[p3]).wait()

    def half(j, k):
        p3 = k % 3
        p2 = k % 2
        rbuf, sidx, sval = rbufs[p3], sidxs[p3], svals[p3]
        ibuf, vbuf = ibufs[p2], vbufs[p2]

        @pl.when(j + 1 < n)
        def _():
            idx_wait(j + 1, (k + 1) % 2)

        @pl.when(j >= 2)
        def _():
            scatter_wait((k + 1) % 3)

        @pl.when(j + 1 < n)
        def _():
            gather_start(j + 1, (k + 1) % 2, (k + 1) % 3)

        gather_wait(p2, p3)
        for g in range(CHUNK // 16):
            sl = pl.ds(g * 16, 16)
            sidx[sl] = ibuf[1, sl]
            sval[sl] = vbuf[sl]

        @pl.when(j + 2 < n)
        def _():
            idx_start(j + 2, k % 2)

        def scale_body(g, _):
            vv = sval[pl.ds(g * 16, 16)]
            for l in range(16):
                v = vv[l]
                e = g * 16 + l
                for d in range(8):
                    sl = pl.ds(d * 16, 16)
                    rbuf[e, sl] = rbuf[e, sl] * v
            return 0

        lax.fori_loop(0, CHUNK // 16, scale_body, 0)
        scatter_start(p3)

    idx_start(0, 0)
    idx_wait(0, 0)
    gather_start(0, 0, 0)
    idx_start(1, 1)

    def chunk_body(i6, _):
        for k in range(6):
            half(6 * i6 + k, k)
        return 0

    lax.fori_loop(0, n // 6, chunk_body, 0)
    scatter_wait((n - 2) % 3)
    scatter_wait((n - 1) % 3)
    plsc.subcore_barrier()

    zb = z_row_base + base_row

    @pl.when(sid < 15)
    def _():
        for k in range(5):
            pltpu.sync_copy(acc.at[pl.ds(base_row + k * CHUNK, CHUNK)],
                            z_hbm.at[pl.ds(zb + k * CHUNK, CHUNK)])
        rem = ROWS_MAIN - 5 * CHUNK
        pltpu.sync_copy(acc.at[pl.ds(base_row + 5 * CHUNK, rem)],
                        z_hbm.at[pl.ds(zb + 5 * CHUNK, rem)])

    @pl.when(sid == 15)
    def _():
        base = 15 * ROWS_MAIN
        zbase = z_row_base + base
        for k in range(5):
            pltpu.sync_copy(acc.at[pl.ds(base + k * CHUNK, CHUNK)],
                            z_hbm.at[pl.ds(zbase + k * CHUNK, CHUNK)])
        rem = ROWS_LAST - 5 * CHUNK
        pltpu.sync_copy(acc.at[pl.ds(base + 5 * CHUNK, rem)],
                        z_hbm.at[pl.ds(zbase + 5 * CHUNK, rem)])

    plsc.subcore_barrier()


@functools.partial(
    pl.kernel,
    mesh=_MESH,
    out_type=jax.ShapeDtypeStruct((4 * N_PAD, DIM), jnp.float32),
    scratch_types=[
        pltpu.VMEM_SHARED((N_S, DIM), jnp.float32),
        pltpu.VMEM((2, CHUNK), jnp.int32),
        pltpu.VMEM((2, CHUNK), jnp.int32),
        pltpu.VMEM((CHUNK,), jnp.float32),
        pltpu.VMEM((CHUNK,), jnp.float32),
        pltpu.VMEM((CHUNK, DIM), jnp.float32),
        pltpu.VMEM((CHUNK, DIM), jnp.float32),
        pltpu.VMEM((CHUNK, DIM), jnp.float32),
        pltpu.VMEM((CHUNK,), jnp.int32),
        pltpu.VMEM((CHUNK,), jnp.int32),
        pltpu.VMEM((CHUNK,), jnp.int32),
        pltpu.VMEM((CHUNK,), jnp.float32),
        pltpu.VMEM((CHUNK,), jnp.float32),
        pltpu.VMEM((CHUNK,), jnp.float32),
        pltpu.SemaphoreType.DMA,
        pltpu.SemaphoreType.DMA,
        pltpu.SemaphoreType.DMA,
        pltpu.SemaphoreType.DMA,
        pltpu.SemaphoreType.DMA,
        pltpu.SemaphoreType.DMA,
        pltpu.SemaphoreType.DMA,
        pltpu.SemaphoreType.DMA,
        pltpu.SemaphoreType.DMA,
        pltpu.SemaphoreType.DMA,
    ],
)
def _sc_pair(p1idx, p1vals, p2idx, p2vals, xz, zflat,
             acc, ibuf0, ibuf1, vbuf0, vbuf1, rbuf0, rbuf1, rbuf2,
             sidx0, sidx1, sidx2, sval0, sval1, sval2,
             isem0, isem1, vsem0, vsem1,
             gsem0, gsem1, gsem2, ssem0, ssem1, ssem2):
    cid = lax.axis_index("c")
    sid = lax.axis_index("s")
    ibufs = (ibuf0, ibuf1)
    vbufs = (vbuf0, vbuf1)
    rbufs = (rbuf0, rbuf1, rbuf2)
    sidxs = (sidx0, sidx1, sidx2)
    svals = (sval0, sval1, sval2)
    isems = (isem0, isem1)
    vsems = (vsem0, vsem1)
    gsems = (gsem0, gsem1, gsem2)
    ssems = (ssem0, ssem1, ssem2)

    def spmm(packed, pvals, x_hbm, chunk_base, z_row_base):
        _do_spmm(packed, pvals, x_hbm, zflat, chunk_base, z_row_base,
                 acc, ibufs, vbufs, rbufs, sidxs, svals,
                 isems, vsems, gsems, ssems, sid)

    spmm(p1idx, p1vals, xz, cid * N_CHUNKS, cid * N_PAD)
    spmm(p2idx, p2vals, zflat, cid * N_CHUNKS, (3 - cid) * N_PAD)


def _pack_edges(rows, cols, vals, x_slot):
    pad = E_PAD - E_EDGES
    c = jnp.pad(cols.astype(jnp.int32) + x_slot * N_PAD,
                (0, pad)).reshape(N_CHUNKS, 1, CHUNK)
    r = jnp.pad(rows.astype(jnp.int32), (0, pad)).reshape(N_CHUNKS, 1, CHUNK)
    v = jnp.pad(vals, (0, pad)).reshape(N_CHUNKS, CHUNK)
    return jnp.concatenate([c, r], axis=1), v


GB_ROWS = B // 32


@functools.partial(
    pl.kernel,
    mesh=_MESH,
    out_type=[jax.ShapeDtypeStruct((B, DIM), jnp.float32)] * 6,
    scratch_types=[
        pltpu.VMEM((GB_ROWS,), jnp.int32),
        pltpu.VMEM((GB_ROWS, DIM), jnp.float32),
        pltpu.SemaphoreType.DMA,
    ],
)
def _sc_gather(g_s, e_s, g_h, e_h, sids, hids, pos, neg,
               gs_sel, es_sel, gh_sel, eh_sel, ehp, ehn,
               idxv, robuf, sem):
    cid = lax.axis_index("c")
    sid = lax.axis_index("s")
    base = (sid * 2 + cid) * GB_ROWS

    def gather(src, idx_hbm, out):
        pltpu.sync_copy(idx_hbm.at[pl.ds(base, GB_ROWS)], idxv)
        pltpu.async_copy(src.at[idxv], robuf, sem).wait()
        pltpu.sync_copy(robuf, out.at[pl.ds(base, GB_ROWS)])

    gather(g_s, sids, gs_sel)
    gather(e_s, sids, es_sel)
    gather(g_h, hids, gh_sel)
    gather(e_h, hids, eh_sel)
    gather(e_h, pos, ehp)
    gather(e_h, neg, ehn)


def _slot_spec(slot):
    return pl.BlockSpec((MB, DIM), lambda k, s=slot: (k + s * SPB, 0))


def _w2_kernel(vt_ref, ut_ref, eh0_ref, zh1_ref, es0_ref, zs1_ref,
               ws_ref, wh_ref):
    @pl.when(pl.program_id(0) == 0)
    def _():
        ws_ref[...] = jnp.zeros_like(ws_ref)
        wh_ref[...] = jnp.zeros_like(wh_ref)

    ws_ref[...] += jnp.dot(vt_ref[...], eh0_ref[...] + zh1_ref[...],
                           preferred_element_type=jnp.float32)
    wh_ref[...] += jnp.dot(ut_ref[...], es0_ref[...] + zs1_ref[...],
                           preferred_element_type=jnp.float32)


def _lowrank_w(vt_pad, ut_pad, xz, zflat):
    w_spec = pl.BlockSpec((RANK, DIM), lambda k: (0, 0))
    return pl.pallas_call(
        _w2_kernel,
        grid=(SPB,),
        in_specs=[
            pl.BlockSpec((RANK, MB), lambda k: (0, k)),
            pl.BlockSpec((RANK, MB), lambda k: (0, k)),
            _slot_spec(1),
            _slot_spec(1),
            _slot_spec(0),
            _slot_spec(0),
        ],
        out_specs=[w_spec, w_spec],
        out_shape=[jax.ShapeDtypeStruct((RANK, DIM), jnp.float32)] * 2,
    )(vt_pad, ut_pad, xz, zflat, xz, zflat)


def _sums_adj_kernel(es0, zs1, zs2, eh0, zh1, zh2, ess0, ehh0,
                     ums, vms, ws, wh,
                     g_s, g_h, e_s, e_h, sqn):
    @pl.when(pl.program_id(0) == 0)
    def _():
        sqn[0, 0] = 0.0

    e_s[...] = es0[...] + zs1[...] + zs2[...]
    e_h[...] = eh0[...] + zh1[...] + zh2[...]
    g_s[...] = es0[...] + jnp.dot(ums[...], ws[...],
                                  preferred_element_type=jnp.float32)
    g_h[...] = eh0[...] + jnp.dot(vms[...], wh[...],
                                  preferred_element_type=jnp.float32)
    sqn[0, 0] += (jnp.sum(es0[...] ** 2) + jnp.sum(eh0[...] ** 2)
                  + jnp.sum(ess0[...] ** 2) + jnp.sum(ehh0[...] ** 2))


def _sums_adj(xz, zadj, ums, vms, ws, wh):
    row_spec = pl.BlockSpec((MB, DIM), lambda k: (k, 0))
    rank_spec = pl.BlockSpec((MB, RANK), lambda k: (k, 0))
    w_spec = pl.BlockSpec((RANK, DIM), lambda k: (0, 0))
    shape = jax.ShapeDtypeStruct((N_PAD, DIM), jnp.float32)
    in_specs = [
        _slot_spec(0), _slot_spec(0), _slot_spec(2),
        _slot_spec(1), _slot_spec(1), _slot_spec(3),
        _slot_spec(2), _slot_spec(3),
        rank_spec, rank_spec, w_spec, w_spec,
    ]
    return pl.pallas_call(
        _sums_adj_kernel,
        grid=(SPB,),
        in_specs=in_specs,
        out_specs=[row_spec] * 4 + [
            pl.BlockSpec(memory_space=pltpu.SMEM)],
        out_shape=[shape] * 4 + [
            jax.ShapeDtypeStruct((1, 1), jnp.float32)],
    )(xz, zadj, zadj, xz, zadj, zadj, xz, xz, ums, vms, ws, wh)


def _sums_ab_kernel(e_s, e_h, ess0, zss1, zss2, ehh0, zhh1, zhh2,
                    a_out, b_out):
    a_out[...] = e_s[...] + ess0[...] + zss1[...] + zss2[...]
    b_out[...] = e_h[...] + ehh0[...] + zhh1[...] + zhh2[...]


def _sums_ab(e_s, e_h, xz, zsshh):
    row_spec = pl.BlockSpec((MB, DIM), lambda k: (k, 0))
    shape = jax.ShapeDtypeStruct((N_PAD, DIM), jnp.float32)
    in_specs = [
        row_spec, row_spec,
        _slot_spec(2), _slot_spec(0), _slot_spec(3),
        _slot_spec(3), _slot_spec(1), _slot_spec(2),
    ]
    return pl.pallas_call(
        _sums_ab_kernel,
        grid=(SPB,),
        in_specs=in_specs,
        out_specs=[row_spec] * 2,
        out_shape=[shape] * 2,
    )(e_s, e_h, xz, zsshh, zsshh, xz, zsshh, zsshh)


def _esynd_kernel(ps_ref, a_ref, g8_ref, b8_ref, e_ref, psum):
    k = pl.program_id(0)

    @pl.when(k == 0)
    def _():
        e_ref[...] = jnp.zeros_like(e_ref)
        psum[...] = jnp.zeros_like(psum)

    e_ref[...] += jnp.dot(ps_ref[...], a_ref[...],
                          preferred_element_type=jnp.float32)
    psum[...] += jnp.sum(ps_ref[...], axis=1, keepdims=True)

    @pl.when(k == pl.num_programs(0) - 1)
    def _():
        e = e_ref[...] / psum[...]
        mean = jnp.mean(e, axis=0, keepdims=True)
        var = jnp.mean(e * e, axis=0, keepdims=True) - mean * mean
        e = (e - mean) / jnp.sqrt(var + BN_EPS)
        e = e * g8_ref[0:1, :] + b8_ref[0:1, :]
        e_ref[...] = jnp.maximum(e, 0.0)


def _esynd_bn(ps_pad, a_pad, gamma8, beta8):
    return pl.pallas_call(
        _esynd_kernel,
        grid=(N_PAD // MB,),
        in_specs=[
            pl.BlockSpec((BU, MB), lambda k: (0, k)),
            pl.BlockSpec((MB, DIM), lambda k: (k, 0)),
            pl.BlockSpec((8, DIM), lambda k: (0, 0)),
            pl.BlockSpec((8, DIM), lambda k: (0, 0)),
        ],
        out_specs=pl.BlockSpec((BU, DIM), lambda k: (0, 0)),
        out_shape=jax.ShapeDtypeStruct((BU, DIM), jnp.float32),
        scratch_shapes=[pltpu.VMEM((BU, 1), jnp.float32)],
    )(ps_pad, a_pad, gamma8, beta8)


def _pre_kernel(e_ref, b_ref, o_ref):
    o_ref[...] = jax.lax.dot_general(
        e_ref[...], b_ref[...], (((1,), (1,)), ((), ())),
        preferred_element_type=jnp.float32)


def _pre_matmul(e_bn, b_pad):
    return pl.pallas_call(
        _pre_kernel,
        grid=(N_PAD // MB,),
        in_specs=[
            pl.BlockSpec((BU, DIM), lambda k: (0, 0)),
            pl.BlockSpec((MB, DIM), lambda k: (k, 0)),
        ],
        out_specs=pl.BlockSpec((BU, MB), lambda k: (0, k)),
        out_shape=jax.ShapeDtypeStruct((BU, N_PAD), jnp.float32),
    )(e_bn, b_pad)


def _scores_kernel(gs_ref, es_sel_ref, esb_ref, gh_ref, eh_sel_ref, ehb_ref,
                   ehp_ref, ehn_ref,
                   dens_ref, denh_ref, digs_ref, digh_ref, bpp_ref, bpn_ref):
    k = pl.program_id(0)

    @pl.when(k == 0)
    def _():
        dens_ref[...] = jnp.zeros_like(dens_ref)
        denh_ref[...] = jnp.zeros_like(denh_ref)
        digs_ref[...] = jnp.sum(gs_ref[...] * es_sel_ref[...], axis=1,
                                keepdims=True)
        digh_ref[...] = jnp.sum(gh_ref[...] * eh_sel_ref[...], axis=1,
                                keepdims=True)
        bpp_ref[...] = jnp.sum(es_sel_ref[...] * ehp_ref[...], axis=1,
                               keepdims=True)
        bpn_ref[...] = jnp.sum(es_sel_ref[...] * ehn_ref[...], axis=1,
                               keepdims=True)

    ss = jax.lax.dot_general(gs_ref[...], esb_ref[...],
                             (((1,), (1,)), ((), ())),
                             preferred_element_type=jnp.float32)
    dens_ref[...] += jnp.sum(jnp.exp(ss / TEMP), axis=1, keepdims=True)
    sh = jax.lax.dot_general(gh_ref[...], ehb_ref[...],
                             (((1,), (1,)), ((), ())),
                             preferred_element_type=jnp.float32)
    denh_ref[...] += jnp.sum(jnp.exp(sh / TEMP), axis=1, keepdims=True)

    @pl.when(k == pl.num_programs(0) - 1)
    def _():
        dens_ref[...] -= jnp.float32(N_PAD - N_S)
        denh_ref[...] -= jnp.float32(N_PAD - N_S)


def _scores(gs_sel, es_sel, es_pad, gh_sel, eh_sel, eh_pad, ehp, ehn):
    vec = jax.ShapeDtypeStruct((B, 1), jnp.float32)
    sel_spec = pl.BlockSpec((B, DIM), lambda k: (0, 0))
    blk_spec = pl.BlockSpec((MB, DIM), lambda k: (k, 0))
    out_spec = pl.BlockSpec((B, 1), lambda k: (0, 0))
    return pl.pallas_call(
        _scores_kernel,
        grid=(N_PAD // MB,),
        in_specs=[sel_spec, sel_spec, blk_spec, sel_spec, sel_spec,
                  blk_spec, sel_spec, sel_spec],
        out_specs=[out_spec] * 6,
        out_shape=[vec] * 6,
    )(gs_sel, es_sel, es_pad, gh_sel, eh_sel, eh_pad, ehp, ehn)


def _final_kernel(dens_ref, denh_ref, digs_ref, digh_ref, bpp_ref, bpn_ref,
                  sqn8_ref, g8_ref, b8_ref,
                  loss_ref, lossr_ref, losss_ref):
    neg = (jnp.mean(jnp.log(dens_ref[...] + 1e-08))
           + jnp.mean(jnp.log(denh_ref[...] + 1e-08)))
    pos = (jnp.mean(jnp.clip(digs_ref[...] / TEMP, -5.0, 5.0))
           + jnp.mean(jnp.clip(digh_ref[...] / TEMP, -5.0, 5.0)))
    loss_s = LAMBDA_1 * (neg - pos)
    diff = bpp_ref[...] - bpn_ref[...]
    loss_r = -jnp.mean(jnp.log(jax.nn.sigmoid(diff)))
    reg = (sqn8_ref[0, 0] + jnp.sum(g8_ref[0:1, :] ** 2)
           + jnp.sum(b8_ref[0:1, :] ** 2)) * LAMBDA_2
    loss_ref[0, 0] = loss_r + loss_s + reg
    lossr_ref[0, 0] = loss_r
    losss_ref[0, 0] = loss_s


def _finalize(dens, denh, digs, digh, bpp, bpn, sqn8, gamma8, beta8):
    vec_spec = pl.BlockSpec((B, 1), lambda: (0, 0))
    row_spec = pl.BlockSpec((8, DIM), lambda: (0, 0))
    scal = jax.ShapeDtypeStruct((1, 1), jnp.float32)
    return pl.pallas_call(
        _final_kernel,
        grid=(),
        in_specs=[vec_spec] * 6 + [row_spec] * 3,
        out_specs=[pl.BlockSpec(memory_space=pltpu.SMEM)] * 3,
        out_shape=[scal] * 3,
    )(dens, denh, digs, digh, bpp, bpn, sqn8, gamma8, beta8)


def kernel(sids, hids, pos, neg, ps, E_s_0, E_h_0, E_ss_0, E_hh_0,
           adj_rows, adj_cols, adj_vals, ss_rows, ss_cols, ss_vals,
           hh_rows, hh_cols, hh_vals, u_mul_s, vt, v_mul_s, ut,
           bn_gamma, bn_beta):
    zpad = jnp.zeros((N_PAD - N_S, DIM), jnp.float32)
    xz = jnp.concatenate([E_s_0, zpad, E_h_0, zpad, E_ss_0, zpad,
                          E_hh_0, zpad], axis=0)
    a1_uses = [
        _pack_edges(adj_rows, adj_cols, adj_vals, 1),
        _pack_edges(adj_cols, adj_rows, adj_vals, 0),
    ]
    a2_uses = [
        _pack_edges(adj_cols, adj_rows, adj_vals, 0),
        _pack_edges(adj_rows, adj_cols, adj_vals, 1),
    ]
    b1_uses = [
        _pack_edges(ss_rows, ss_cols, ss_vals, 2),
        _pack_edges(hh_rows, hh_cols, hh_vals, 3),
    ]
    b2_uses = [
        _pack_edges(ss_rows, ss_cols, ss_vals, 0),
        _pack_edges(hh_rows, hh_cols, hh_vals, 1),
    ]

    def cat(uses, i):
        return jnp.concatenate([u[i] for u in uses], axis=0)

    zadj = _sc_pair(cat(a1_uses, 0), cat(a1_uses, 1),
                    cat(a2_uses, 0), cat(a2_uses, 1), xz)
    zsshh = _sc_pair(cat(b1_uses, 0), cat(b1_uses, 1),
                     cat(b2_uses, 0), cat(b2_uses, 1), xz)

    vt_pad = jnp.pad(vt, ((0, 0), (0, N_PAD - N_S)))
    ut_pad = jnp.pad(ut, ((0, 0), (0, N_PAD - N_S)))
    w_s, w_h = _lowrank_w(vt_pad, ut_pad, xz, zadj)

    ums = jnp.pad(u_mul_s, ((0, N_PAD - N_S), (0, 0)))
    vms = jnp.pad(v_mul_s, ((0, N_PAD - N_S), (0, 0)))
    g_s, g_h, e_s, e_h, sqn = _sums_adj(xz, zadj, ums, vms, w_s, w_h)
    a_sum, b_sum = _sums_ab(e_s, e_h, xz, zsshh)

    gamma8 = jnp.broadcast_to(bn_gamma[None, :], (8, DIM))
    beta8 = jnp.broadcast_to(bn_beta[None, :], (8, DIM))
    ps_pad = jnp.pad(ps, ((0, 0), (0, N_PAD - N_S)))
    e_bn = _esynd_bn(ps_pad, a_sum, gamma8, beta8)
    pre = _pre_matmul(e_bn, b_sum)[:, :N_S]

    gs_sel, es_sel, gh_sel, eh_sel, ehp, ehn = _sc_gather(
        g_s, e_s, g_h, e_h, sids.astype(jnp.int32), hids.astype(jnp.int32),
        pos.astype(jnp.int32), neg.astype(jnp.int32))

    dens, denh, digs, digh, bpp, bpn = _scores(
        gs_sel, es_sel, e_s, gh_sel, eh_sel, e_h, ehp, ehn)

    sqn8 = jnp.broadcast_to(sqn, (8, DIM))
    loss, loss_r, loss_s = _finalize(dens, denh, digs, digh, bpp, bpn,
                                     sqn8, gamma8, beta8)
    return (loss.reshape(()), loss_r.reshape(()), loss_s.reshape(()), pre)
